# baseline (speedup 1.0000x reference)
import jax
import jax.numpy as jnp
from jax import lax
from jax.experimental import pallas as pl
from jax.experimental.pallas import tpu as pltpu
from jax.experimental.pallas import tpu_sc as plsc

VOCAB = 1000000
EMB = 32
B = 4096
L = 50
NPAIR = 5
NUNIT = NPAIR * L
NROW = VOCAB * EMB // 128

VC = 512
NCH = VOCAB // VC
VTAIL = VOCAB - (VOCAB // VC) * VC
TROWS = VTAIL * EMB // 128
OR = VC * EMB // 128

BC = 128
NBC = B // BC

NC, NS = 2, 16
NW = NC * NS

_params = pltpu.CompilerParams(needs_layout_passes=False)


def _tr_kernel(wct, wbt, tailc, tailb, outc, outb, s0, s1, o0, o1,
               isem0, isem1, wsem0, wsem1):
    wid = lax.axis_index("s") * NC + lax.axis_index("c")
    iota = lax.iota(jnp.int32, 16)
    gv_a = lax.shift_right_logical(iota, 3)
    gv_b = gv_a + 2
    er_v = lax.bitwise_and(iota, 7)

    def table(w2, out):
        nu = (NCH - wid + NW - 1) // NW

        def stage(v0, vc, sbuf, sem):
            for g in range(4):
                pltpu.async_copy(w2.at[pl.ds(g * 8, 8), pl.ds(v0, vc)],
                                 sbuf.at[g, :, pl.ds(0, vc)], sem)

        def drain_stage(vc, sem):
            for g in range(4):
                pltpu.make_async_copy(
                    w2.at[pl.ds(0, 8), pl.ds(0, vc)],
                    s0.at[0, :, pl.ds(0, vc)], sem).wait()

        def transpose(vc, sbuf, ob):
            def tbody(r, carry):
                base = iota * 0 + r * 4
                for k in range(4):
                    vv = base + k
                    va = plsc.load_gather(sbuf, [gv_a, er_v, vv])
                    vb = plsc.load_gather(sbuf, [gv_b, er_v, vv])
                    ob[r, pl.ds(k * EMB, 16)] = va
                    ob[r, pl.ds(k * EMB + 16, 16)] = vb
                return carry

            lax.fori_loop(0, vc // 4, tbody, 0, unroll=False)

        def drain_write(rows, sem):
            pltpu.make_async_copy(
                o0.at[pl.ds(0, rows), :], out.at[pl.ds(0, rows), :],
                sem).wait()

        def chunk_i(i):
            return pl.multiple_of((wid + i * NW) * VC, 128)

        stage(chunk_i(0), VC, s0, isem0)

        def body(k, carry):
            for half, sbuf, ob, isem, wsem, sb_n, isem_n in (
                    (0, s0, o0, isem0, wsem0, s1, isem1),
                    (1, s1, o1, isem1, wsem1, s0, isem0)):
                i = 2 * k + half

                @pl.when(i < nu)
                def _():
                    drain_stage(VC, isem)

                    @pl.when(i + 1 < nu)
                    def _():
                        stage(chunk_i(i + 1), VC, sb_n, isem_n)

                    @pl.when(i >= 2)
                    def _():
                        drain_write(OR, wsem)

                    transpose(VC, sbuf, ob)
                    pltpu.async_copy(
                        ob,
                        out.at[pl.ds(pl.multiple_of(
                            (wid + i * NW) * OR, 8), OR), :],
                        wsem)
            return carry

        lax.fori_loop(0, (nu + 1) // 2, body, 0, unroll=False)
        drain_write(OR, wsem0)
        drain_write(OR, wsem1)

    table(wct, outc)
    table(wbt, outb)

    for tw, tail, out in ((0, tailc, outc), (1, tailb, outb)):
        @pl.when(wid == tw)
        def _():
            pltpu.async_copy(
                tail, out.at[pl.ds(NCH * VC * EMB // 128, TROWS), :], wsem0)
            pltpu.make_async_copy(
                tail, out.at[pl.ds(0, TROWS), :], wsem0).wait()


def _lk_kernel(idx_hbm, wc, wb, out_hbm,
               idx_a, idx_b, ric0, ric1, ric2, rib0, rib1, rib2,
               ac0, ac1, ac2, ab0, ab1, ab2, dc0, dc1, dc2, db0, db1, db2,
               ob0, ob1, isem, gsem0, gsem1, gsem2, wsem0, wsem1):
    wid = lax.axis_index("s") * NC + lax.axis_index("c")
    iota = lax.iota(jnp.int32, 16)
    rics = (ric0, ric1, ric2)
    ribs = (rib0, rib1, rib2)
    acs = (ac0, ac1, ac2)
    abs_ = (ab0, ab1, ab2)
    dcs = (dc0, dc1, dc2)
    dbs = (db0, db1, db2)
    gsems = (gsem0, gsem1, gsem2)
    obs = (ob0, ob1)
    wsems = (wsem0, wsem1)

    def do_unit(u, carry):
        p = u // L
        l = u % L
        ia = pltpu.async_copy(
            idx_hbm.at[2 * p, pl.ds(pl.multiple_of(l * 32, 8), 32), :],
            idx_a, isem)
        ib = pltpu.async_copy(
            idx_hbm.at[2 * p + 1, pl.ds(pl.multiple_of(l * 32, 8), 32), :],
            idx_b, isem)
        ia.wait()
        ib.wait()

        def prep(q, s):
            ric, rib, ac, ab = rics[s], ribs[s], acs[s], abs_[s]

            def pbody(j, carry):
                vc = idx_a[q, pl.ds(j * 16, 16)]
                vb = idx_b[q, pl.ds(j * 16, 16)]
                ric[pl.ds(j * 16, 16)] = lax.shift_right_logical(vc, 2)
                rib[pl.ds(j * 16, 16)] = lax.shift_right_logical(vb, 2)
                ac[pl.ds(j * 16, 16)] = lax.shift_left(
                    lax.bitwise_and(vc, 3), 5)
                ab[pl.ds(j * 16, 16)] = lax.shift_left(
                    lax.bitwise_and(vb, 3), 5)
                return carry

            lax.fori_loop(0, BC // 16, pbody, 0, unroll=False)

        def fire(s):
            pltpu.async_copy(wc.at[rics[s]], dcs[s], gsems[s])
            pltpu.async_copy(wb.at[ribs[s]], dbs[s], gsems[s])

        def drain_gathers(s):
            pltpu.make_async_copy(wc.at[pl.ds(0, BC), :], dc0, gsems[s]).wait()
            pltpu.make_async_copy(wc.at[pl.ds(0, BC), :], dc0, gsems[s]).wait()

        def extract(s, w):
            ac, ab, dc, db, ob = acs[s], abs_[s], dcs[s], dbs[s], obs[w]

            def ebody(j, carry):
                rv = iota + j * 16
                va = ac[pl.ds(j * 16, 16)]
                vb = ab[pl.ds(j * 16, 16)]
                for e in range(EMB):
                    ob[e, pl.ds(j * 16, 16)] = plsc.load_gather(
                        dc, [rv, va + e])
                for e in range(EMB):
                    ob[EMB + e, pl.ds(j * 16, 16)] = plsc.load_gather(
                        db, [rv, vb + e])
                return carry

            lax.fori_loop(0, BC // 16, ebody, 0, unroll=False)

        def write(q, w):
            pltpu.async_copy(
                obs[w],
                out_hbm.at[p, l, :, pl.ds(pl.multiple_of(q * BC, 128), BC)],
                wsems[w])

        def drain_write(w):
            pltpu.make_async_copy(
                ob0, out_hbm.at[0, 0, :, pl.ds(0, BC)], wsems[w]).wait()

        for q0 in range(3):
            prep(q0, q0)
            fire(q0)

        def six(m, carry):
            for qq in range(6):
                q = 6 * m + qq
                s, w = qq % 3, qq % 2
                drain_gathers(s)
                if qq >= 2:
                    drain_write(w)
                else:
                    @pl.when(m >= 1)
                    def _():
                        drain_write(w)
                extract(s, w)

                @pl.when(q + 3 < NBC)
                def _():
                    prep(q + 3, s)
                    fire(s)

                write(q, w)
            return carry

        lax.fori_loop(0, 5, six, 0, unroll=False)
        for q in range(30, NBC):
            s, w = q % 3, q % 2
            drain_gathers(s)
            drain_write(w)
            extract(s, w)
            write(q, w)
        drain_write(0)
        drain_write(1)
        return carry

    nu = (NUNIT - wid + NW - 1) // NW

    def body(k, carry):
        return do_unit(wid + k * NW, carry)

    lax.fori_loop(0, nu, body, 0, unroll=False)


@jax.jit
def _run(idx_all, wct, wbt, tailc, tailb):
    mesh = plsc.VectorSubcoreMesh(core_axis_name="c", subcore_axis_name="s")
    tr = pl.kernel(
        _tr_kernel,
        out_type=(jax.ShapeDtypeStruct((NROW, 128), jnp.float32),
                  jax.ShapeDtypeStruct((NROW, 128), jnp.float32)),
        mesh=mesh,
        scratch_types=[
            pltpu.VMEM((4, 8, VC), jnp.float32),
            pltpu.VMEM((4, 8, VC), jnp.float32),
            pltpu.VMEM((OR, 128), jnp.float32),
            pltpu.VMEM((OR, 128), jnp.float32),
            pltpu.SemaphoreType.DMA,
            pltpu.SemaphoreType.DMA,
            pltpu.SemaphoreType.DMA,
            pltpu.SemaphoreType.DMA,
        ],
        compiler_params=_params,
    )
    wc, wb = tr(wct, wbt, tailc, tailb)
    lk = pl.kernel(
        _lk_kernel,
        out_type=jax.ShapeDtypeStruct((NPAIR, L, 2 * EMB, B), jnp.float32),
        mesh=mesh,
        scratch_types=(
            [pltpu.VMEM((32, BC), jnp.int32)] * 2
            + [pltpu.VMEM((BC,), jnp.int32)] * 12
            + [pltpu.VMEM((BC, 128), jnp.float32)] * 6
            + [pltpu.VMEM((2 * EMB, BC), jnp.float32)] * 2
            + [pltpu.SemaphoreType.DMA] * 6
        ),
        compiler_params=_params,
    )
    return lk(idx_all, wc, wb)


def kernel(idx0, idx1, idx2, idx3, idx4, idx5, idx6, idx7, idx8, idx9,
           W_cate, W_brand):
    idx_all = jnp.stack(
        [i.T.reshape(L * 32, 128) for i in
         (idx0, idx1, idx2, idx3, idx4, idx5, idx6, idx7, idx8, idx9)])
    tailc = W_cate[NCH * VC:, :].reshape(TROWS, 128)
    tailb = W_brand[NCH * VC:, :].reshape(TROWS, 128)
    out_k = _run(idx_all, W_cate.T, W_brand.T, tailc, tailb)
    return jnp.transpose(out_k, (0, 3, 1, 2))
